# Initial kernel scaffold; baseline (speedup 1.0000x reference)
import jax
import jax.numpy as jnp
from jax.experimental import pallas as pl


def kernel(x_zoom7, idx, group_idx, embeddings, W, b):
    raise NotImplementedError("write your pallas kernel here")


# speedup vs baseline: 2.3355x; 2.3355x over previous
import functools

import jax
import jax.numpy as jnp
from jax import lax
from jax.experimental import pallas as pl
from jax.experimental.pallas import tpu as pltpu
from jax.experimental.pallas import tpu_sc as plsc

GW = 128


def _sc_gather(table, idx2d):
    info = plsc.get_sparse_core_info()
    nc, ns = info.num_cores, info.num_subcores
    nw = nc * ns
    groups, gw = idx2d.shape
    feat = table.shape[1]
    g_per_w = groups // nw
    mesh = plsc.VectorSubcoreMesh(core_axis_name="c", subcore_axis_name="s")

    @functools.partial(
        pl.kernel, mesh=mesh,
        compiler_params=pltpu.CompilerParams(use_tc_tiling_on_sc=False),
        out_type=jax.ShapeDtypeStruct((groups * gw, feat), jnp.float32),
        scratch_types=[
            pltpu.VMEM((g_per_w, gw), jnp.int32),
            pltpu.VMEM((gw, feat), jnp.float32),
            pltpu.SemaphoreType.DMA,
        ],
    )
    def k(table_hbm, idx_hbm, out_hbm, idx_v, row_v, sem):
        wid = lax.axis_index("s") * nc + lax.axis_index("c")
        gbase = wid * g_per_w
        pltpu.sync_copy(idx_hbm.at[pl.ds(gbase, g_per_w)], idx_v)

        def body(j, carry):
            pltpu.async_copy(table_hbm.at[idx_v.at[j]], row_v, sem).wait()
            pltpu.sync_copy(row_v, out_hbm.at[pl.ds((gbase + j) * gw, gw)])
            return carry

        lax.fori_loop(0, g_per_w, body, 0)

    return k(table, idx2d)


def _tc_film(g, x, W, b):
    rows, feat = g.shape
    blk = 1024

    def body(g_ref, x_ref, w_ref, b_ref, o_ref):
        h = jnp.dot(g_ref[...], w_ref[...],
                    preferred_element_type=jnp.float32) + b_ref[...]
        o_ref[...] = x_ref[...] * h[:, :feat] + h[:, feat:]

    return pl.pallas_call(
        body,
        grid=(rows // blk,),
        in_specs=[
            pl.BlockSpec((blk, feat), lambda i: (i, 0)),
            pl.BlockSpec((blk, feat), lambda i: (i, 0)),
            pl.BlockSpec((feat, 2 * feat), lambda i: (0, 0)),
            pl.BlockSpec((1, 2 * feat), lambda i: (0, 0)),
        ],
        out_specs=pl.BlockSpec((blk, feat), lambda i: (i, 0)),
        out_shape=jax.ShapeDtypeStruct((rows, feat), jnp.float32),
    )(g, x, W, b)


def kernel(x_zoom7, idx, group_idx, embeddings, W, b):
    feat = x_zoom7.shape[-1]
    table = embeddings[0]
    idx2d = idx.reshape(-1, GW)
    gathered = _sc_gather(table, idx2d)
    x_flat = x_zoom7.reshape(-1, feat)
    out = _tc_film(gathered, x_flat, W, b.reshape(1, -1))
    return out.reshape(x_zoom7.shape)


# speedup vs baseline: 3.0849x; 1.3208x over previous
import functools

import jax
import jax.numpy as jnp
from jax import lax
from jax.experimental import pallas as pl
from jax.experimental.pallas import tpu as pltpu
from jax.experimental.pallas import tpu_sc as plsc

GW = 128
NBUF = 4


def _tc_precompute(table, W, b):
    n, feat = table.shape
    blk = 2048

    def body(t_ref, w_ref, b_ref, o_ref):
        o_ref[...] = jnp.dot(t_ref[...], w_ref[...],
                             preferred_element_type=jnp.float32) + b_ref[...]

    return pl.pallas_call(
        body,
        grid=(n // blk,),
        in_specs=[
            pl.BlockSpec((blk, feat), lambda i: (i, 0)),
            pl.BlockSpec((feat, 2 * feat), lambda i: (0, 0)),
            pl.BlockSpec((1, 2 * feat), lambda i: (0, 0)),
        ],
        out_specs=pl.BlockSpec((blk, 2 * feat), lambda i: (i, 0)),
        out_shape=jax.ShapeDtypeStruct((n, 2 * feat), jnp.float32),
    )(table, W, b)


def _sc_gather(t, idx2d):
    info = plsc.get_sparse_core_info()
    nc, ns = info.num_cores, info.num_subcores
    nw = nc * ns
    groups, gw = idx2d.shape
    width = t.shape[1]
    g_per_w = groups // nw
    mesh = plsc.VectorSubcoreMesh(core_axis_name="c", subcore_axis_name="s")

    @functools.partial(
        pl.kernel, mesh=mesh,
        out_type=jax.ShapeDtypeStruct((groups * gw, width), jnp.float32),
        scratch_types=[
            pltpu.VMEM((g_per_w, gw), jnp.int32),
            [pltpu.VMEM((gw, width), jnp.float32) for _ in range(NBUF)],
            [pltpu.SemaphoreType.DMA for _ in range(NBUF)],
        ],
    )
    def k(t_hbm, idx_hbm, out_hbm, idx_v, bufs, sems):
        wid = lax.axis_index("s") * nc + lax.axis_index("c")
        gbase = wid * g_per_w
        pltpu.sync_copy(idx_hbm.at[pl.ds(gbase, g_per_w)], idx_v)

        def start(j, b):
            pltpu.async_copy(t_hbm.at[idx_v.at[j]], bufs[b], sems[b])

        def finish(j, b):
            pltpu.make_async_copy(t_hbm.at[idx_v.at[j]], bufs[b],
                                  sems[b]).wait()
            pltpu.sync_copy(bufs[b], out_hbm.at[pl.ds((gbase + j) * gw, gw)])

        for b in range(NBUF):
            start(b, b)

        def body(j0, carry):
            for b in range(NBUF):
                j = j0 * NBUF + b
                finish(j, b)
                start(j + NBUF, b)
            return carry

        lax.fori_loop(0, g_per_w // NBUF - 1, body, 0)
        for b in range(NBUF):
            finish(g_per_w - NBUF + b, b)

    return k(t, idx2d)


def _tc_film(g, x):
    rows, feat = x.shape
    blk = 2048

    def body(g_ref, x_ref, o_ref):
        gv = g_ref[...]
        o_ref[...] = x_ref[...] * gv[:, :feat] + gv[:, feat:]

    return pl.pallas_call(
        body,
        grid=(rows // blk,),
        in_specs=[
            pl.BlockSpec((blk, 2 * feat), lambda i: (i, 0)),
            pl.BlockSpec((blk, feat), lambda i: (i, 0)),
        ],
        out_specs=pl.BlockSpec((blk, feat), lambda i: (i, 0)),
        out_shape=jax.ShapeDtypeStruct((rows, feat), jnp.float32),
    )(g, x)


def kernel(x_zoom7, idx, group_idx, embeddings, W, b):
    feat = x_zoom7.shape[-1]
    table = embeddings[0]
    t = _tc_precompute(table, W, b.reshape(1, -1))
    idx2d = idx.reshape(-1, GW)
    gathered = _sc_gather(t, idx2d)
    x_flat = x_zoom7.reshape(-1, feat)
    out = _tc_film(gathered, x_flat)
    return out.reshape(x_zoom7.shape)


# speedup vs baseline: 4.0438x; 1.3108x over previous
import functools

import jax
import jax.numpy as jnp
from jax import lax
from jax.experimental import pallas as pl
from jax.experimental.pallas import tpu as pltpu
from jax.experimental.pallas import tpu_sc as plsc

GW = 128
NBUF = 4


def _tc_precompute(table_t, W, b):
    feat, n = table_t.shape
    blk = 2048

    def body(t_ref, w_ref, b_ref, o_ref):
        o_ref[...] = lax.dot_general(
            t_ref[...], w_ref[...], (((0,), (0,)), ((), ())),
            preferred_element_type=jnp.float32) + b_ref[...]

    return pl.pallas_call(
        body,
        grid=(n // blk,),
        in_specs=[
            pl.BlockSpec((feat, blk), lambda i: (0, i)),
            pl.BlockSpec((feat, 2 * feat), lambda i: (0, 0)),
            pl.BlockSpec((1, 2 * feat), lambda i: (0, 0)),
        ],
        out_specs=pl.BlockSpec((blk, 2 * feat), lambda i: (i, 0)),
        out_shape=jax.ShapeDtypeStruct((n, 2 * feat), jnp.float32),
    )(table_t, W, b)


def _sc_gather(t, idx2d):
    info = plsc.get_sparse_core_info()
    nc, ns = info.num_cores, info.num_subcores
    nw = nc * ns
    groups, gw = idx2d.shape
    width = t.shape[1]
    g_per_w = groups // nw
    mesh = plsc.VectorSubcoreMesh(core_axis_name="c", subcore_axis_name="s")

    @functools.partial(
        pl.kernel, mesh=mesh,
        out_type=jax.ShapeDtypeStruct((groups * gw, width), jnp.float32),
        scratch_types=[
            pltpu.VMEM((g_per_w, gw), jnp.int32),
            [pltpu.VMEM((gw, width), jnp.float32) for _ in range(NBUF)],
            [pltpu.SemaphoreType.DMA for _ in range(NBUF)],
        ],
    )
    def k(t_hbm, idx_hbm, out_hbm, idx_v, bufs, sems):
        wid = lax.axis_index("s") * nc + lax.axis_index("c")
        gbase = wid * g_per_w
        pltpu.sync_copy(idx_hbm.at[pl.ds(gbase, g_per_w)], idx_v)

        def start(j, b):
            pltpu.async_copy(t_hbm.at[idx_v.at[j]], bufs[b], sems[b])

        def finish(j, b):
            pltpu.make_async_copy(t_hbm.at[idx_v.at[j]], bufs[b],
                                  sems[b]).wait()
            pltpu.sync_copy(bufs[b], out_hbm.at[pl.ds((gbase + j) * gw, gw)])

        for b in range(NBUF):
            start(b, b)

        def body(j0, carry):
            for b in range(NBUF):
                j = j0 * NBUF + b
                finish(j, b)
                start(j + NBUF, b)
            return carry

        lax.fori_loop(0, g_per_w // NBUF - 1, body, 0)
        for b in range(NBUF):
            finish(g_per_w - NBUF + b, b)

    return k(t, idx2d)


def _tc_film(g, x3):
    nb, feat, p = x3.shape
    blk = 2048
    jblocks = p // blk

    def body(g_ref, x_ref, o_ref):
        gv = g_ref[...]
        scale = jnp.transpose(gv[:, :feat])
        shift = jnp.transpose(gv[:, feat:])
        o_ref[0] = x_ref[0] * scale + shift

    return pl.pallas_call(
        body,
        grid=(nb, jblocks),
        in_specs=[
            pl.BlockSpec((blk, 2 * feat), lambda b, j: (b * jblocks + j, 0)),
            pl.BlockSpec((1, feat, blk), lambda b, j: (b, 0, j)),
        ],
        out_specs=pl.BlockSpec((1, feat, blk), lambda b, j: (b, 0, j)),
        out_shape=jax.ShapeDtypeStruct((nb, feat, p), jnp.float32),
    )(g, x3)


def kernel(x_zoom7, idx, group_idx, embeddings, W, b):
    nb, _, _, p, feat = x_zoom7.shape
    table_t = jnp.transpose(embeddings, (0, 2, 1))[0]
    t = _tc_precompute(table_t, W, b.reshape(1, -1))
    idx2d = idx.reshape(-1, GW)
    gathered = _sc_gather(t, idx2d)
    x3 = jnp.transpose(x_zoom7, (0, 1, 2, 4, 3)).reshape(nb, feat, p)
    out3 = _tc_film(gathered, x3)
    return jnp.transpose(out3.reshape(nb, 1, 1, feat, p), (0, 1, 2, 4, 3))
